# Initial kernel scaffold; baseline (speedup 1.0000x reference)
import jax
import jax.numpy as jnp
from jax.experimental import pallas as pl


def kernel(verts, bary_weights, v_index):
    raise NotImplementedError("write your pallas kernel here")


# speedup vs baseline: 7.0677x; 7.0677x over previous
import functools

import jax
import jax.numpy as jnp
from jax import lax
from jax.experimental import pallas as pl
from jax.experimental.pallas import tpu as pltpu
from jax.experimental.pallas import tpu_sc as plsc

B = 16
NV = 6890
H = 512
W = 512
C = 3
P = H * W

_info = plsc.get_sparse_core_info()
NC = _info.num_cores
NS = _info.num_subcores
L = _info.num_lanes
NW = NC * NS
PPW = P // NW
NVP = ((NV * C + 15) // 16) * 16


def _sc_body(verts_hbm, idx_hbm, bary_hbm, out_hbm, idx_v, bary_v, vbuf, obuf):
    wid = lax.axis_index("s") * NC + lax.axis_index("c")
    base_px = wid * PPW

    pltpu.sync_copy(idx_hbm.at[:, pl.ds(base_px, PPW)], idx_v)
    pltpu.sync_copy(bary_hbm.at[:, pl.ds(base_px, PPW)], bary_v)

    iota3 = lax.iota(jnp.int32, L) * 3

    def px_body(i, _):
        s = i * L
        i0 = idx_v[0, pl.ds(s, L)]
        i1 = idx_v[1, pl.ds(s, L)]
        i2 = idx_v[2, pl.ds(s, L)]
        b0 = bary_v[0, pl.ds(s, L)]
        b1 = bary_v[1, pl.ds(s, L)]
        b2 = bary_v[2, pl.ds(s, L)]
        for c in range(C):
            g0 = plsc.load_gather(vbuf, [i0 + c])
            g1 = plsc.load_gather(vbuf, [i1 + c])
            g2 = plsc.load_gather(vbuf, [i2 + c])
            acc = b0 * g0 + b1 * g1 + b2 * g2
            plsc.store_scatter(obuf, [iota3 + (i * (L * C) + c)], acc)
        return _

    for b in range(B):
        pltpu.sync_copy(verts_hbm.at[b], vbuf)
        lax.fori_loop(0, PPW // L, px_body, 0, unroll=2)
        pltpu.sync_copy(obuf, out_hbm.at[b, pl.ds(base_px * C, PPW * C)])


@functools.partial(jax.jit, static_argnames=())
def kernel(verts, bary_weights, v_index):
    idx3 = (v_index.reshape(P, C).astype(jnp.int32) * 3).T
    bary = bary_weights.reshape(P, C).T
    verts_flat = jnp.pad(
        verts.reshape(B, NV * C), ((0, 0), (0, NVP - NV * C))
    )

    sc = pl.kernel(
        _sc_body,
        mesh=plsc.VectorSubcoreMesh(core_axis_name="c", subcore_axis_name="s"),
        out_type=jax.ShapeDtypeStruct((B, P * C), jnp.float32),
        compiler_params=pltpu.CompilerParams(needs_layout_passes=False),
        scratch_types=[
            pltpu.VMEM((C, PPW), jnp.int32),
            pltpu.VMEM((C, PPW), jnp.float32),
            pltpu.VMEM((NVP,), jnp.float32),
            pltpu.VMEM((PPW * C,), jnp.float32),
        ],
    )
    out = sc(verts_flat, idx3, bary)
    return out.reshape(B, H, W, C)
